# baseline (speedup 1.0000x reference)
import jax
import jax.numpy as jnp
import numpy as np
from jax import lax
from jax.experimental import pallas as pl
from jax.experimental.pallas import tpu as pltpu
from jax.experimental.pallas import tpu_sc as plsc

_B, _H, _W = 256, 512, 512
_FLIP_IDX = [
    2, 3, 4, 5, 6, 7, 8, 9, 10, 11, 15, 16, 18, 19, 20, 22, 24, 29, 30,
    31, 32, 34, 35, 37, 39, 42, 43, 44, 45, 49, 50, 53, 54, 56, 58, 61,
    63, 65, 67, 69, 70, 72, 77, 78, 80, 81, 82, 83, 85, 90, 92, 94, 96,
    99, 101, 102, 106, 108, 110, 111, 112, 114, 117, 118, 121, 123, 128,
    129, 130, 135, 137, 138, 139, 140, 142, 144, 147, 148, 152, 153, 154,
    155, 156, 157, 159, 160, 163, 167, 169, 173, 174, 175, 176, 177, 178,
    179, 183, 184, 185, 186, 188, 189, 191, 192, 195, 197, 199, 200, 211,
    212, 217, 218, 219, 223, 233, 234, 235, 236, 237, 239, 240, 241, 245,
    246, 249, 251, 253, 254,
]
_MASK1D = np.zeros((_B,), np.float32)
_MASK1D[np.asarray(_FLIP_IDX)] = 1.0

_L = 16
_RCH = 32
_NCH = _H // _RCH
_NBUF = 6
_NW = 32
_MASK_ROWS = np.repeat(_MASK1D[:, None], _L, axis=1)


def _sc_build(nb):
    bpw = nb // _NW
    nunits = bpw * _NCH
    mesh = plsc.VectorSubcoreMesh(core_axis_name="c", subcore_axis_name="s")

    def body(x_hbm, m_hbm, o_hbm, buf0, buf1, buf2, buf3, buf4, buf5,
             mbuf, in_sem, out_sem, m_sem):
        bufs = (buf0, buf1, buf2, buf3, buf4, buf5)
        wid = lax.axis_index("s") * 2 + lax.axis_index("c")
        base = wid * bpw
        pltpu.async_copy(m_hbm.at[pl.ds(base, bpw)], mbuf, m_sem).wait()

        def compute(k, j):
            mv = mbuf[j]
            bk = bufs[k]

            @plsc.parallel_loop(0, _RCH * (_W // _L), step=1, unroll=8)
            def _(i):
                r = lax.shift_right_logical(i, 5)
                cc = pl.multiple_of(
                    lax.shift_left(lax.bitwise_and(i, 31), 4), _L)
                bk[r, pl.ds(cc, _L)] = jnp.abs(mv - bk[r, pl.ds(cc, _L)])

        def in_copy(u, k):
            j, c = divmod(u, _NCH)
            return pltpu.async_copy(
                x_hbm.at[base + j, pl.ds(c * _RCH, _RCH), :], bufs[k], in_sem)

        def out_copy(u, k):
            j, c = divmod(u, _NCH)
            return pltpu.async_copy(
                bufs[k], o_hbm.at[base + j, 0, pl.ds(c * _RCH, _RCH), :], out_sem)

        ins = {0: in_copy(0, 0)}
        outs = {}
        for u in range(nunits):
            k = u % _NBUF
            if u + 1 < nunits:
                prev = u + 1 - _NBUF
                if prev >= 0:
                    outs.pop(prev).wait()
                ins[u + 1] = in_copy(u + 1, (u + 1) % _NBUF)
            ins.pop(u).wait()
            compute(k, u // _NCH)
            outs[u] = out_copy(u, k)
        for u in sorted(outs):
            outs.pop(u).wait()

    return pl.kernel(
        body,
        out_type=jax.ShapeDtypeStruct((nb, 1, _H, _W), jnp.float32),
        mesh=mesh,
        scratch_types=[
            pltpu.VMEM((_RCH, _W), jnp.float32),
            pltpu.VMEM((_RCH, _W), jnp.float32),
            pltpu.VMEM((_RCH, _W), jnp.float32),
            pltpu.VMEM((_RCH, _W), jnp.float32),
            pltpu.VMEM((_RCH, _W), jnp.float32),
            pltpu.VMEM((_RCH, _W), jnp.float32),
            pltpu.VMEM((bpw, _L), jnp.float32),
            pltpu.SemaphoreType.DMA,
            pltpu.SemaphoreType.DMA,
            pltpu.SemaphoreType.DMA,
        ],
    )


_sc_call = _sc_build(_B)


def kernel(inp):
    mtab = jnp.asarray(_MASK_ROWS)
    return _sc_call(inp, mtab)


# speedup vs baseline: 1.0289x; 1.0289x over previous
import jax
import jax.numpy as jnp
import numpy as np
from jax import lax
from jax.experimental import pallas as pl
from jax.experimental.pallas import tpu as pltpu
from jax.experimental.pallas import tpu_sc as plsc

_B, _H, _W = 256, 512, 512
_FLIP_IDX = [
    2, 3, 4, 5, 6, 7, 8, 9, 10, 11, 15, 16, 18, 19, 20, 22, 24, 29, 30,
    31, 32, 34, 35, 37, 39, 42, 43, 44, 45, 49, 50, 53, 54, 56, 58, 61,
    63, 65, 67, 69, 70, 72, 77, 78, 80, 81, 82, 83, 85, 90, 92, 94, 96,
    99, 101, 102, 106, 108, 110, 111, 112, 114, 117, 118, 121, 123, 128,
    129, 130, 135, 137, 138, 139, 140, 142, 144, 147, 148, 152, 153, 154,
    155, 156, 157, 159, 160, 163, 167, 169, 173, 174, 175, 176, 177, 178,
    179, 183, 184, 185, 186, 188, 189, 191, 192, 195, 197, 199, 200, 211,
    212, 217, 218, 219, 223, 233, 234, 235, 236, 237, 239, 240, 241, 245,
    246, 249, 251, 253, 254,
]
_MASK1D = np.zeros((_B,), np.float32)
_MASK1D[np.asarray(_FLIP_IDX)] = 1.0

_L = 16
_RCH = 64
_NCH = _H // _RCH
_NBUF = 3
_NW = 32


def _sc_build(nb):
    bpw = nb // _NW
    nunits = bpw * _NCH
    wbits = [
        sum(int(_MASK1D[w * bpw + j]) << w for w in range(_NW))
        for j in range(bpw)
    ]
    mesh = plsc.VectorSubcoreMesh(core_axis_name="c", subcore_axis_name="s")

    def body(x_hbm, o_hbm, buf0, buf1, buf2, in_sem, out_sem):
        bufs = (buf0, buf1, buf2)
        wid = lax.axis_index("s") * 2 + lax.axis_index("c")
        base = wid * bpw
        widu = wid.astype(jnp.uint32)

        def compute(k, j):
            bk = bufs[k]
            flip = lax.bitwise_and(
                lax.shift_right_logical(jnp.uint32(wbits[j]), widu),
                jnp.uint32(1))

            @pl.when(flip == jnp.uint32(1))
            def _():
                @plsc.parallel_loop(0, _RCH * (_W // _L), step=1, unroll=8)
                def _(i):
                    r = lax.shift_right_logical(i, 5)
                    cc = pl.multiple_of(
                        lax.shift_left(lax.bitwise_and(i, 31), 4), _L)
                    bk[r, pl.ds(cc, _L)] = jnp.abs(1.0 - bk[r, pl.ds(cc, _L)])

        def in_copy(u, k):
            j, c = divmod(u, _NCH)
            return pltpu.async_copy(
                x_hbm.at[base + j, pl.ds(c * _RCH, _RCH), :], bufs[k], in_sem)

        def out_copy(u, k):
            j, c = divmod(u, _NCH)
            return pltpu.async_copy(
                bufs[k], o_hbm.at[base + j, 0, pl.ds(c * _RCH, _RCH), :], out_sem)

        ins = {0: in_copy(0, 0)}
        outs = {}
        for u in range(nunits):
            k = u % _NBUF
            if u + 1 < nunits:
                prev = u + 1 - _NBUF
                if prev >= 0:
                    outs.pop(prev).wait()
                ins[u + 1] = in_copy(u + 1, (u + 1) % _NBUF)
            ins.pop(u).wait()
            compute(k, u // _NCH)
            outs[u] = out_copy(u, k)
        for u in sorted(outs):
            outs.pop(u).wait()

    return pl.kernel(
        body,
        out_type=jax.ShapeDtypeStruct((nb, 1, _H, _W), jnp.float32),
        mesh=mesh,
        scratch_types=[
            pltpu.VMEM((_RCH, _W), jnp.float32),
            pltpu.VMEM((_RCH, _W), jnp.float32),
            pltpu.VMEM((_RCH, _W), jnp.float32),
            pltpu.SemaphoreType.DMA,
            pltpu.SemaphoreType.DMA,
        ],
    )


_sc_call = _sc_build(_B)


def kernel(inp):
    return _sc_call(inp)
